# initial kernel scaffold (unmeasured)
import jax
import jax.numpy as jnp
from jax import lax
from jax.experimental import pallas as pl
from jax.experimental.pallas import tpu as pltpu


def kernel(x, pi):
    def body(x_ref, pi_ref, out_ref, send_sem, recv_sem):
        my_i = lax.axis_index("i")
        dst = pi_ref[my_i]
        rdma = pltpu.make_async_remote_copy(
            src_ref=x_ref,
            dst_ref=out_ref,
            send_sem=send_sem,
            recv_sem=recv_sem,
            device_id=(dst,),
            device_id_type=pl.DeviceIdType.MESH,
        )
        rdma.start()
        rdma.wait()

    out_shape = jax.ShapeDtypeStruct(x.shape, jnp.float32)
    return pl.pallas_call(
        body,
        out_shape=out_shape,
        in_specs=[
            pl.BlockSpec(memory_space=pltpu.VMEM),
            pl.BlockSpec(memory_space=pltpu.SMEM),
        ],
        out_specs=pl.BlockSpec(memory_space=pltpu.VMEM),
        scratch_shapes=[
            pltpu.SemaphoreType.DMA,
            pltpu.SemaphoreType.DMA,
        ],
        compiler_params=pltpu.CompilerParams(collective_id=0),
    )(x, pi)


# baseline (device time: 201248 ns/iter reference)
import jax
import jax.numpy as jnp
from jax import lax
from jax.experimental import pallas as pl
from jax.experimental.pallas import tpu as pltpu


def kernel(x, pi):
    def body(x_ref, pi_ref, out_ref, send_sem, recv_sem):
        my_i = lax.axis_index("i")
        dst = pi_ref[my_i]
        rdma = pltpu.make_async_remote_copy(
            src_ref=x_ref,
            dst_ref=out_ref,
            send_sem=send_sem,
            recv_sem=recv_sem,
            device_id=(dst,),
            device_id_type=pl.DeviceIdType.MESH,
        )
        rdma.start()
        rdma.wait()

    out_shape = jax.ShapeDtypeStruct(x.shape, jnp.float32)
    return pl.pallas_call(
        body,
        out_shape=out_shape,
        in_specs=[
            pl.BlockSpec(memory_space=pltpu.VMEM),
            pl.BlockSpec(memory_space=pltpu.SMEM),
        ],
        out_specs=pl.BlockSpec(memory_space=pltpu.VMEM),
        scratch_shapes=[
            pltpu.SemaphoreType.DMA,
            pltpu.SemaphoreType.DMA,
        ],
    )(x, pi)


# device time: 160620 ns/iter; 1.2529x vs baseline; 1.2529x over previous
import jax
import jax.numpy as jnp
from jax import lax
from jax.experimental import pallas as pl
from jax.experimental.pallas import tpu as pltpu

N_DEV = 32


def kernel(x, pi):
    _, m, n = x.shape
    mh = m // 2

    def body(x_ref, pi_ref, out_ref, relay_ref,
             s_dir, r_dir, s_rel, r_rel, s_fwd, r_fwd):
        my_i = lax.axis_index("i")
        dst = pi_ref[my_i]
        relay = lax.rem(dst + N_DEV - 1, N_DEV)
        right = lax.rem(my_i + 1, N_DEV)

        rdma_dir = pltpu.make_async_remote_copy(
            src_ref=x_ref.at[:, pl.ds(0, mh), :],
            dst_ref=out_ref.at[:, pl.ds(0, mh), :],
            send_sem=s_dir,
            recv_sem=r_dir,
            device_id=(dst,),
            device_id_type=pl.DeviceIdType.MESH,
        )
        rdma_dir.start()

        rdma_rel = pltpu.make_async_remote_copy(
            src_ref=x_ref.at[:, pl.ds(mh, mh), :],
            dst_ref=relay_ref,
            send_sem=s_rel,
            recv_sem=r_rel,
            device_id=(relay,),
            device_id_type=pl.DeviceIdType.MESH,
        )
        rdma_rel.start()

        rdma_rel.wait_recv()
        rdma_fwd = pltpu.make_async_remote_copy(
            src_ref=relay_ref,
            dst_ref=out_ref.at[:, pl.ds(mh, mh), :],
            send_sem=s_fwd,
            recv_sem=r_fwd,
            device_id=(right,),
            device_id_type=pl.DeviceIdType.MESH,
        )
        rdma_fwd.start()

        rdma_dir.wait()
        rdma_rel.wait_send()
        rdma_fwd.wait()

    out_shape = jax.ShapeDtypeStruct(x.shape, jnp.float32)
    return pl.pallas_call(
        body,
        out_shape=out_shape,
        in_specs=[
            pl.BlockSpec(memory_space=pltpu.VMEM),
            pl.BlockSpec(memory_space=pltpu.SMEM),
        ],
        out_specs=pl.BlockSpec(memory_space=pltpu.VMEM),
        scratch_shapes=[
            pltpu.VMEM((1, mh, n), jnp.float32),
            pltpu.SemaphoreType.DMA,
            pltpu.SemaphoreType.DMA,
            pltpu.SemaphoreType.DMA,
            pltpu.SemaphoreType.DMA,
            pltpu.SemaphoreType.DMA,
            pltpu.SemaphoreType.DMA,
        ],
    )(x, pi)


# device time: 160385 ns/iter; 1.2548x vs baseline; 1.0015x over previous
import jax
import jax.numpy as jnp
from jax import lax
from jax.experimental import pallas as pl
from jax.experimental.pallas import tpu as pltpu

N_DEV = 32
N_CHUNK = 4


def kernel(x, pi):
    _, m, n = x.shape
    mh = m // 2
    rows = mh // N_CHUNK

    def body(x_ref, pi_ref, out_ref, relay_ref,
             s_dir, r_dir, s_rel, r_rel, s_fwd, r_fwd):
        my_i = lax.axis_index("i")
        dst = pi_ref[my_i]
        relay = lax.rem(dst + N_DEV - 1, N_DEV)
        right = lax.rem(my_i + 1, N_DEV)

        rdma_dir = pltpu.make_async_remote_copy(
            src_ref=x_ref.at[:, pl.ds(0, mh), :],
            dst_ref=out_ref.at[:, pl.ds(0, mh), :],
            send_sem=s_dir,
            recv_sem=r_dir,
            device_id=(dst,),
            device_id_type=pl.DeviceIdType.MESH,
        )
        rdma_dir.start()

        rdma_rel = []
        for c in range(N_CHUNK):
            r = pltpu.make_async_remote_copy(
                src_ref=x_ref.at[:, pl.ds(mh + c * rows, rows), :],
                dst_ref=relay_ref.at[:, pl.ds(c * rows, rows), :],
                send_sem=s_rel.at[c],
                recv_sem=r_rel.at[c],
                device_id=(relay,),
                device_id_type=pl.DeviceIdType.MESH,
            )
            r.start()
            rdma_rel.append(r)

        rdma_fwd = []
        for c in range(N_CHUNK):
            rdma_rel[c].wait_recv()
            f = pltpu.make_async_remote_copy(
                src_ref=relay_ref.at[:, pl.ds(c * rows, rows), :],
                dst_ref=out_ref.at[:, pl.ds(mh + c * rows, rows), :],
                send_sem=s_fwd.at[c],
                recv_sem=r_fwd.at[c],
                device_id=(right,),
                device_id_type=pl.DeviceIdType.MESH,
            )
            f.start()
            rdma_fwd.append(f)

        rdma_dir.wait()
        for c in range(N_CHUNK):
            rdma_rel[c].wait_send()
            rdma_fwd[c].wait()

    out_shape = jax.ShapeDtypeStruct(x.shape, jnp.float32)
    return pl.pallas_call(
        body,
        out_shape=out_shape,
        in_specs=[
            pl.BlockSpec(memory_space=pltpu.VMEM),
            pl.BlockSpec(memory_space=pltpu.SMEM),
        ],
        out_specs=pl.BlockSpec(memory_space=pltpu.VMEM),
        scratch_shapes=[
            pltpu.VMEM((1, mh, n), jnp.float32),
            pltpu.SemaphoreType.DMA,
            pltpu.SemaphoreType.DMA,
            pltpu.SemaphoreType.DMA((N_CHUNK,)),
            pltpu.SemaphoreType.DMA((N_CHUNK,)),
            pltpu.SemaphoreType.DMA((N_CHUNK,)),
            pltpu.SemaphoreType.DMA((N_CHUNK,)),
        ],
    )(x, pi)


# device time: 156484 ns/iter; 1.2861x vs baseline; 1.0249x over previous
import jax
import jax.numpy as jnp
from jax import lax
from jax.experimental import pallas as pl
from jax.experimental.pallas import tpu as pltpu

N_DEV = 32
N_CHUNK = 4


def kernel(x, pi):
    _, m, n = x.shape
    mh = m // 2
    rows = mh // N_CHUNK

    def body(x_ref, pi_ref, out_ref, relay_ref,
             s_dir, r_dir, s_rel, r_rel, s_fwd, r_fwd):
        my_i = lax.axis_index("i")
        dst = pi_ref[my_i]
        relay = lax.rem(my_i + 1, N_DEV)
        fwd_dst = lax.rem(dst + N_DEV - 1, N_DEV)

        rdma_dir = pltpu.make_async_remote_copy(
            src_ref=x_ref.at[:, pl.ds(0, mh), :],
            dst_ref=out_ref.at[:, pl.ds(0, mh), :],
            send_sem=s_dir,
            recv_sem=r_dir,
            device_id=(dst,),
            device_id_type=pl.DeviceIdType.MESH,
        )
        rdma_dir.start()

        rdma_rel = []
        for c in range(N_CHUNK):
            r = pltpu.make_async_remote_copy(
                src_ref=x_ref.at[:, pl.ds(mh + c * rows, rows), :],
                dst_ref=relay_ref.at[:, pl.ds(c * rows, rows), :],
                send_sem=s_rel.at[c],
                recv_sem=r_rel.at[c],
                device_id=(relay,),
                device_id_type=pl.DeviceIdType.MESH,
            )
            r.start()
            rdma_rel.append(r)

        rdma_fwd = []
        for c in range(N_CHUNK):
            rdma_rel[c].wait_recv()
            f = pltpu.make_async_remote_copy(
                src_ref=relay_ref.at[:, pl.ds(c * rows, rows), :],
                dst_ref=out_ref.at[:, pl.ds(mh + c * rows, rows), :],
                send_sem=s_fwd.at[c],
                recv_sem=r_fwd.at[c],
                device_id=(fwd_dst,),
                device_id_type=pl.DeviceIdType.MESH,
            )
            f.start()
            rdma_fwd.append(f)

        rdma_dir.wait()
        for c in range(N_CHUNK):
            rdma_rel[c].wait_send()
            rdma_fwd[c].wait()

    out_shape = jax.ShapeDtypeStruct(x.shape, jnp.float32)
    return pl.pallas_call(
        body,
        out_shape=out_shape,
        in_specs=[
            pl.BlockSpec(memory_space=pltpu.VMEM),
            pl.BlockSpec(memory_space=pltpu.SMEM),
        ],
        out_specs=pl.BlockSpec(memory_space=pltpu.VMEM),
        scratch_shapes=[
            pltpu.VMEM((1, mh, n), jnp.float32),
            pltpu.SemaphoreType.DMA,
            pltpu.SemaphoreType.DMA,
            pltpu.SemaphoreType.DMA((N_CHUNK,)),
            pltpu.SemaphoreType.DMA((N_CHUNK,)),
            pltpu.SemaphoreType.DMA((N_CHUNK,)),
            pltpu.SemaphoreType.DMA((N_CHUNK,)),
        ],
    )(x, pi)
